# Initial kernel scaffold; baseline (speedup 1.0000x reference)
import jax
import jax.numpy as jnp
from jax.experimental import pallas as pl


def kernel(position, edge_index):
    raise NotImplementedError("write your pallas kernel here")


# speedup vs baseline: 32.2551x; 32.2551x over previous
import functools

import jax
import jax.numpy as jnp
from jax import lax
from jax.experimental import pallas as pl
from jax.experimental.pallas import tpu as pltpu
from jax.experimental.pallas import tpu_sc as plsc

N_NODES = 100000
N_EDGES = 3200000
NC = 2
NS = 16
NW = NC * NS
PER_TILE = N_EDGES // NW
C = 2000
NCHUNK = PER_TILE // C
G = C // 16

_MAGIC = 0x5F3759DF


def _rsqrt(x):
    i = lax.bitcast_convert_type(x, jnp.int32)
    y = lax.bitcast_convert_type(
        jnp.int32(_MAGIC) - lax.shift_right_arithmetic(i, 1), jnp.float32)
    half_x = x * jnp.float32(0.5)
    for _ in range(3):
        y = y * (jnp.float32(1.5) - half_x * y * y)
    return y


def _edge_kernel(table_hbm, ei_hbm, out_hbm, idx2, idxw, rows, out_v, sem_g):
    wid = lax.axis_index("s") * NC + lax.axis_index("c")
    base0 = wid * PER_TILE

    iota = lax.iota(jnp.int32, 16)

    def chunk(g, _):
        base = base0 + g * C
        pltpu.sync_copy(ei_hbm.at[pl.ds(base, C)], idx2.at[pl.ds(0, C)])
        pltpu.sync_copy(ei_hbm.at[pl.ds(N_EDGES + base, C)],
                        idx2.at[pl.ds(C, C)])

        def mkidx(k, _):
            o = k * 16
            vs = idx2[pl.ds(o, 16)] * 4
            vd = idx2[pl.ds(C + o, 16)] * 4
            idxw[pl.ds(o, 16)] = vs
            idxw[pl.ds(C + o, 16)] = vs + 1
            idxw[pl.ds(2 * C + o, 16)] = vs + 2
            idxw[pl.ds(3 * C + o, 16)] = vd
            idxw[pl.ds(4 * C + o, 16)] = vd + 1
            idxw[pl.ds(5 * C + o, 16)] = vd + 2
            return ()

        lax.fori_loop(0, G, mkidx, (), unroll=2)
        pltpu.async_copy(table_hbm.at[idxw], rows, sem_g).wait()

        def group(k, _):
            o = k * 16
            dx = rows[pl.ds(o, 16)] - rows[pl.ds(3 * C + o, 16)]
            dy = rows[pl.ds(C + o, 16)] - rows[pl.ds(4 * C + o, 16)]
            dz = rows[pl.ds(2 * C + o, 16)] - rows[pl.ds(5 * C + o, 16)]
            d2 = dx * dx + dy * dy + dz * dz
            dist = d2 * _rsqrt(d2)
            ob = iota * 4 + k * 64
            plsc.store_scatter(out_v, [ob], dx)
            plsc.store_scatter(out_v, [ob + 1], dy)
            plsc.store_scatter(out_v, [ob + 2], dz)
            plsc.store_scatter(out_v, [ob + 3], dist)
            return ()

        lax.fori_loop(0, G, group, (), unroll=2)
        pltpu.sync_copy(out_v, out_hbm.at[pl.ds(base * 4, C * 4)])
        return ()

    lax.fori_loop(0, NCHUNK, chunk, ())


def _run(table_flat, ei_flat):
    mesh = plsc.VectorSubcoreMesh(core_axis_name="c", subcore_axis_name="s")
    f = functools.partial(
        pl.kernel,
        mesh=mesh,
        out_type=jax.ShapeDtypeStruct((N_EDGES * 4,), jnp.float32),
        scratch_types=[
            pltpu.VMEM((2 * C,), jnp.int32),
            pltpu.VMEM((6 * C,), jnp.int32),
            pltpu.VMEM((6 * C,), jnp.float32),
            pltpu.VMEM((4 * C,), jnp.float32),
            pltpu.SemaphoreType.DMA,
        ],
        compiler_params=pltpu.CompilerParams(needs_layout_passes=False),
    )(_edge_kernel)
    return f(table_flat, ei_flat)


def kernel(position, edge_index):
    pos_last = position[:, -1, :]
    table_flat = jnp.pad(pos_last, ((0, 0), (0, 1))).reshape(-1)
    ei_flat = edge_index.reshape(-1)
    edge_attr = _run(table_flat, ei_flat).reshape(N_EDGES, 4)
    node_attr_flat = position.reshape(position.shape[0], -1)
    return (node_attr_flat, edge_index, edge_attr)
